# baseline (speedup 1.0000x reference)
import functools

import numpy as np
import jax
import jax.numpy as jnp
from jax import lax
from jax.experimental import pallas as pl
from jax.experimental.pallas import tpu as pltpu
from jax.experimental.pallas import tpu_sc as plsc

_D_MODEL = 64
_MAX_LEN = 512


def _position_embedding_np(max_len, d_model):
    position = np.arange(0, max_len, dtype=np.float64)[:, None]
    div_term = np.exp(
        -np.arange(0, d_model, 2, dtype=np.float64) * (np.log(10000.0) / d_model)
    )
    pe = np.zeros((max_len, d_model), dtype=np.float32)
    pe[:, 0::2] = np.sin(position * div_term)
    pe[:, 1::2] = np.cos(position * div_term)
    return pe


_PE_NP = _position_embedding_np(_MAX_LEN, _D_MODEL)


@functools.cache
def _build(B, S, D):
    info = plsc.get_sparse_core_info()
    NC, NS = info.num_cores, info.num_subcores
    NW = NC * NS
    assert B % NW == 0
    R = B // NW
    K = 2
    assert R % K == 0
    G = R // K
    splits = []
    off = 0
    while off < S:
        n = min(128, S - off)
        splits.append((off, n))
        off += n
    scale = float(np.sqrt(np.float32(D)))
    mesh = plsc.VectorSubcoreMesh(core_axis_name="c", subcore_axis_name="s")

    @functools.partial(
        pl.kernel,
        mesh=mesh,
        compiler_params=pltpu.CompilerParams(use_tc_tiling_on_sc=False),
        out_type=jax.ShapeDtypeStruct((B, S, D), jnp.float32),
        scratch_types=[
            pltpu.VMEM((S, D), jnp.float32),
            [pltpu.VMEM((S,), jnp.int32) for _ in range(K)],
            pltpu.VMEM((K, S, D), jnp.float32),
            pltpu.SemaphoreType.DMA,
        ],
    )
    def sc_kernel(x_hbm, table_hbm, pe_hbm, out_hbm, pe_v, idx_v, buf, sem):
        wid = lax.axis_index("s") * NC + lax.axis_index("c")
        r0 = wid * R
        pltpu.sync_copy(pe_hbm, pe_v)

        def group(g, carry):
            row = r0 + g * K
            for kk in range(K):
                pltpu.sync_copy(x_hbm.at[row + kk], idx_v[kk])
            cps = []
            for kk in range(K):
                for off, n in splits:
                    cps.append(
                        pltpu.async_copy(
                            table_hbm.at[idx_v[kk].at[pl.ds(off, n)]],
                            buf.at[kk, pl.ds(off, n)],
                            sem,
                        )
                    )
            for cp in cps:
                cp.wait()

            def pos(j, c2):
                for dd in range(D // 16):
                    pev = pe_v[j, pl.ds(dd * 16, 16)]
                    for kk in range(K):
                        v = buf[kk, j, pl.ds(dd * 16, 16)]
                        buf[kk, j, pl.ds(dd * 16, 16)] = v * scale + pev
                return c2

            lax.fori_loop(0, S, pos, 0, unroll=False)
            pltpu.sync_copy(buf, out_hbm.at[pl.ds(row, K)])
            return carry

        lax.fori_loop(0, G, group, 0, unroll=False)

    return sc_kernel


def kernel(x, table):
    B, S = x.shape
    D = table.shape[1]
    pe = jnp.asarray(_PE_NP[:S])
    return _build(B, S, D)(x.astype(jnp.int32), table, pe)
